# baseline (speedup 1.0000x reference)
import jax
import jax.numpy as jnp
from jax import lax
from jax.experimental import pallas as pl
from jax.experimental.pallas import tpu as pltpu

_N = 5460
_PAD = 5504
_C = 21
_B = 8
_NOBJ = 8
_NSEL = 49
_REGIONS = (
    (0, 4096, ((0, 4096, 9),)),
    (4096, 1024, ((0, 1024, 9),)),
    (5120, 256, ((0, 256, 9),)),
    (5376, 128, ((0, 64, 9), (64, 80, 9), (80, 84, 4))),
)


def _assignment(locs_ref, pri_ref, boxes_ref, lvals_ref,
                lab_s, num_s, den_s):
    px = pri_ref[0:1, :][None]
    py = pri_ref[1:2, :][None]
    pw = pri_ref[2:3, :][None]
    ph = pri_ref[3:4, :][None]

    bx0 = boxes_ref[:, :, 0:1]
    by0 = boxes_ref[:, :, 1:2]
    bx1 = boxes_ref[:, :, 2:3]
    by1 = boxes_ref[:, :, 3:4]
    cbx = (bx0 + bx1) / 2.0
    cby = (by0 + by1) / 2.0

    dist = jnp.sqrt((cbx - px) ** 2 + (cby - py) ** 2)

    qx0 = px - pw / 2.0
    qy0 = py - ph / 2.0
    qx1 = px + pw / 2.0
    qy1 = py + ph / 2.0

    ltx = jnp.maximum(bx0, qx0)
    lty = jnp.maximum(by0, qy0)
    rbx = jnp.minimum(bx1, qx1)
    rby = jnp.minimum(by1, qy1)
    inter = jnp.clip(rbx - ltx, 0.0, None) * jnp.clip(rby - lty, 0.0, None)
    area_a = (bx1 - bx0) * (by1 - by0)
    area_b = (qx1 - qx0) * (qy1 - qy0)
    ov = inter / jnp.clip(area_a + area_b - inter, 1e-9, None)

    chains = []
    for (start, width, subs) in _REGIONS:
        d_sl = dist[:, :, start:start + width]
        col = lax.broadcasted_iota(jnp.int32, (1, 1, width), 2)
        for (lo, hi, k) in subs:
            lvmask = (col >= lo) & (col < hi)
            d = jnp.where(lvmask, d_sl, jnp.inf)
            chains.append({"start": start, "k": k, "col": col,
                           "lvmask": lvmask, "d": d})
    for t in range(9):
        for ch in chains:
            if t >= ch["k"]:
                continue
            d = ch["d"]
            col = ch["col"]
            rm = jnp.min(d, axis=2, keepdims=True)
            cand = jnp.where(d == rm, col, jnp.int32(2 ** 30))
            idx = jnp.min(cand, axis=2, keepdims=True)
            ch["d"] = jnp.where(col == idx, jnp.inf, d)
    region_sel = {}
    for ch in chains:
        s = jnp.isinf(ch["d"]) & ch["lvmask"]
        key = ch["start"]
        region_sel[key] = (jnp.logical_or(region_sel[key], s)
                          if key in region_sel else s)
    sel = jnp.concatenate([region_sel[s] for (s, _, _) in _REGIONS], axis=2)

    self32 = sel.astype(jnp.float32)
    mean = jnp.sum(ov * self32, axis=2, keepdims=True) / float(_NSEL)
    dev = (ov - mean) * self32
    var = jnp.sum(dev * dev, axis=2, keepdims=True) / float(_NSEL - 1)
    thr = mean + jnp.sqrt(var)

    inside = (bx0 <= px) & (px <= bx1) & (by0 <= py) & (py <= by1)
    m = sel & (ov > thr) & inside

    li = lvals_ref[:, :, :].astype(jnp.int32)
    obcode = lax.broadcasted_iota(jnp.int32, (1, _NOBJ, 1), 1) * 32 + li
    q = jnp.where(m, obcode, 0)
    labcode = jnp.max(q, axis=1)
    lab_s[:, :] = (labcode[:, :_N] & 31).astype(jnp.float32)

    g0 = locs_ref[:, 0:1, :]
    g1 = locs_ref[:, 1:2, :]
    g2 = locs_ref[:, 2:3, :]
    g3 = locs_ref[:, 3:4, :]
    dcx = g0 * pw / 10.0 + px
    dcy = g1 * ph / 10.0 + py
    dw = jnp.exp(g2 / 5.0) * pw
    dh = jnp.exp(g3 / 5.0) * ph
    dx0 = dcx - dw / 2.0
    dy0 = dcy - dh / 2.0
    dx1 = dcx + dw / 2.0
    dy1 = dcy + dh / 2.0

    ltx2 = jnp.maximum(dx0, bx0)
    lty2 = jnp.maximum(dy0, by0)
    rbx2 = jnp.minimum(dx1, bx1)
    rby2 = jnp.minimum(dy1, by1)
    inter2 = (jnp.clip(rbx2 - ltx2, 0.0, None)
              * jnp.clip(rby2 - lty2, 0.0, None))
    ap = (dx1 - dx0) * (dy1 - dy0)
    iou2 = inter2 / jnp.clip(ap + area_a - inter2, 1e-9, None)
    cpx = (dx0 + dx1) / 2.0
    cpy = (dy0 + dy1) / 2.0
    d2 = (cpx - cbx) ** 2 + (cpy - cby) ** 2
    ex0 = jnp.minimum(dx0, bx0)
    ey0 = jnp.minimum(dy0, by0)
    ex1 = jnp.maximum(dx1, bx1)
    ey1 = jnp.maximum(dy1, by1)
    c2 = (ex1 - ex0) ** 2 + (ey1 - ey0) ** 2
    dloss = 1.0 - iou2 + d2 / jnp.clip(c2, 1e-9, None)

    mf = m.astype(jnp.float32)
    num_s[:, :] = jnp.sum(dloss * mf).reshape(1, 1)
    den_s[:, :] = jnp.sum(mf).reshape(1, 1)


def _fused_kernel(locs_ref, pri_ref, boxes_ref, lvals_ref, sc_ref,
                  out_ref, lab_s, num_s, den_s):
    _assignment(locs_ref, pri_ref, boxes_ref, lvals_ref,
                lab_s, num_s, den_s)

    cls = lax.broadcasted_iota(jnp.int32, (_C, 1), 0).astype(jnp.float32)
    fsum = jnp.zeros((1, 1), jnp.float32)
    for i in range(_B):
        labv = lab_s[i:i + 1, :]
        s = jnp.swapaxes(sc_ref[i], 0, 1)
        t = (cls == labv).astype(jnp.float32)
        p = jax.nn.sigmoid(s)
        ce = jnp.maximum(s, 0.0) - s * t + jnp.log1p(jnp.exp(-jnp.abs(s)))
        pt = p * t + (1.0 - p) * (1.0 - t)
        w = 0.25 * t + 0.75 * (1.0 - t)
        om = 1.0 - pt
        fsum = fsum + jnp.sum(w * om * om * ce).reshape(1, 1)

    out_ref[:, :] = (fsum * (1.0 / float(_B * _N))
                     + num_s[:, :] / jnp.maximum(den_s[:, :], 1.0))


def _run(predicted_locs, predicted_scores, boxes, labels, priors,
         interpret=False):
    B, N, C = predicted_scores.shape
    locs_t = jnp.pad(jnp.transpose(predicted_locs, (0, 2, 1)),
                     ((0, 0), (0, 0), (0, _PAD - N)))
    pri_t = jnp.pad(jnp.transpose(priors, (1, 0)), ((0, 0), (0, _PAD - N)))
    lvals = labels.astype(jnp.float32)[..., None]

    out = pl.pallas_call(
        _fused_kernel,
        grid=(1,),
        in_specs=[
            pl.BlockSpec((B, 4, _PAD), lambda i: (0, 0, 0)),
            pl.BlockSpec((4, _PAD), lambda i: (0, 0)),
            pl.BlockSpec((B, _NOBJ, 4), lambda i: (0, 0, 0)),
            pl.BlockSpec((B, _NOBJ, 1), lambda i: (0, 0, 0)),
            pl.BlockSpec((B, _N, _C), lambda i: (0, 0, 0)),
        ],
        out_specs=pl.BlockSpec((1, 1), lambda i: (0, 0)),
        out_shape=jax.ShapeDtypeStruct((1, 1), jnp.float32),
        scratch_shapes=[
            pltpu.VMEM((_B, _N), jnp.float32),
            pltpu.VMEM((1, 1), jnp.float32),
            pltpu.VMEM((1, 1), jnp.float32),
        ],
        interpret=interpret,
    )(locs_t, pri_t, boxes, lvals, predicted_scores)
    return out[0, 0]


def kernel(predicted_locs, predicted_scores, boxes, labels, priors):
    return _run(predicted_locs, predicted_scores, boxes, labels, priors)


# speedup vs baseline: 1.3672x; 1.3672x over previous
import jax
import jax.numpy as jnp
from jax import lax
from jax.experimental import pallas as pl
from jax.experimental.pallas import tpu as pltpu

_N = 5460
_PAD = 5504
_C = 21
_B = 8
_NOBJ = 8
_NSEL = 49
_REGIONS = (
    (0, 4096, ((0, 4096, 9),)),
    (4096, 1024, ((0, 1024, 9),)),
    (5120, 256, ((0, 256, 9),)),
    (5376, 128, ((0, 64, 9), (64, 80, 9), (80, 84, 4))),
)


def _assignment(locs_ref, pri_ref, boxes_ref, lvals_ref,
                lab_s, num_s, den_s):
    px = pri_ref[0:1, :][None]
    py = pri_ref[1:2, :][None]
    pw = pri_ref[2:3, :][None]
    ph = pri_ref[3:4, :][None]

    bx0 = boxes_ref[:, :, 0:1]
    by0 = boxes_ref[:, :, 1:2]
    bx1 = boxes_ref[:, :, 2:3]
    by1 = boxes_ref[:, :, 3:4]
    cbx = (bx0 + bx1) / 2.0
    cby = (by0 + by1) / 2.0

    dist = jnp.sqrt((cbx - px) ** 2 + (cby - py) ** 2)

    qx0 = px - pw / 2.0
    qy0 = py - ph / 2.0
    qx1 = px + pw / 2.0
    qy1 = py + ph / 2.0

    ltx = jnp.maximum(bx0, qx0)
    lty = jnp.maximum(by0, qy0)
    rbx = jnp.minimum(bx1, qx1)
    rby = jnp.minimum(by1, qy1)
    inter = jnp.clip(rbx - ltx, 0.0, None) * jnp.clip(rby - lty, 0.0, None)
    area_a = (bx1 - bx0) * (by1 - by0)
    area_b = (qx1 - qx0) * (qy1 - qy0)
    ov = inter / jnp.clip(area_a + area_b - inter, 1e-9, None)

    chains = []
    for (start, width, subs) in _REGIONS:
        d_sl = dist[:, :, start:start + width]
        col = lax.broadcasted_iota(jnp.int32, (1, 1, width), 2)
        for (lo, hi, k) in subs:
            lvmask = (col >= lo) & (col < hi)
            d = jnp.where(lvmask, d_sl, jnp.inf)
            chains.append({"start": start, "k": k, "col": col,
                           "lvmask": lvmask, "d": d})
    for t in range(9):
        for ch in chains:
            if t >= ch["k"]:
                continue
            d = ch["d"]
            col = ch["col"]
            rm = jnp.min(d, axis=2, keepdims=True)
            cand = jnp.where(d == rm, col, jnp.int32(2 ** 30))
            idx = jnp.min(cand, axis=2, keepdims=True)
            ch["d"] = jnp.where(col == idx, jnp.inf, d)
    region_sel = {}
    for ch in chains:
        s = jnp.isinf(ch["d"]) & ch["lvmask"]
        key = ch["start"]
        region_sel[key] = (jnp.logical_or(region_sel[key], s)
                          if key in region_sel else s)
    sel = jnp.concatenate([region_sel[s] for (s, _, _) in _REGIONS], axis=2)

    self32 = sel.astype(jnp.float32)
    mean = jnp.sum(ov * self32, axis=2, keepdims=True) / float(_NSEL)
    dev = (ov - mean) * self32
    var = jnp.sum(dev * dev, axis=2, keepdims=True) / float(_NSEL - 1)
    thr = mean + jnp.sqrt(var)

    inside = (bx0 <= px) & (px <= bx1) & (by0 <= py) & (py <= by1)
    m = sel & (ov > thr) & inside

    li = lvals_ref[:, :, :].astype(jnp.int32)
    obcode = lax.broadcasted_iota(jnp.int32, (1, _NOBJ, 1), 1) * 32 + li
    q = jnp.where(m, obcode, 0)
    labcode = jnp.max(q, axis=1)
    lab_s[:, :] = (labcode[:, :_N] & 31).astype(jnp.float32)

    g0 = locs_ref[:, 0:1, :]
    g1 = locs_ref[:, 1:2, :]
    g2 = locs_ref[:, 2:3, :]
    g3 = locs_ref[:, 3:4, :]
    dcx = g0 * pw / 10.0 + px
    dcy = g1 * ph / 10.0 + py
    dw = jnp.exp(g2 / 5.0) * pw
    dh = jnp.exp(g3 / 5.0) * ph
    dx0 = dcx - dw / 2.0
    dy0 = dcy - dh / 2.0
    dx1 = dcx + dw / 2.0
    dy1 = dcy + dh / 2.0

    ltx2 = jnp.maximum(dx0, bx0)
    lty2 = jnp.maximum(dy0, by0)
    rbx2 = jnp.minimum(dx1, bx1)
    rby2 = jnp.minimum(dy1, by1)
    inter2 = (jnp.clip(rbx2 - ltx2, 0.0, None)
              * jnp.clip(rby2 - lty2, 0.0, None))
    ap = (dx1 - dx0) * (dy1 - dy0)
    iou2 = inter2 / jnp.clip(ap + area_a - inter2, 1e-9, None)
    cpx = (dx0 + dx1) / 2.0
    cpy = (dy0 + dy1) / 2.0
    d2 = (cpx - cbx) ** 2 + (cpy - cby) ** 2
    ex0 = jnp.minimum(dx0, bx0)
    ey0 = jnp.minimum(dy0, by0)
    ex1 = jnp.maximum(dx1, bx1)
    ey1 = jnp.maximum(dy1, by1)
    c2 = (ex1 - ex0) ** 2 + (ey1 - ey0) ** 2
    dloss = 1.0 - iou2 + d2 / jnp.clip(c2, 1e-9, None)

    mf = m.astype(jnp.float32)
    num_s[:, :] = jnp.sum(dloss * mf).reshape(1, 1)
    den_s[:, :] = jnp.sum(mf).reshape(1, 1)


def _fused_kernel(locs_ref, pri_ref, boxes_ref, lvals_ref, sc_ref,
                  out_ref, lab_s, num_s, den_s):
    _assignment(locs_ref, pri_ref, boxes_ref, lvals_ref,
                lab_s, num_s, den_s)

    cls = lax.broadcasted_iota(jnp.int32, (_C, 1), 0).astype(jnp.float32)
    fsum = jnp.zeros((1, 1), jnp.float32)
    for i in range(_B):
        labv = lab_s[i:i + 1, :]
        s = sc_ref[i]
        t = (cls == labv).astype(jnp.float32)
        p = jax.nn.sigmoid(s)
        ce = jnp.maximum(s, 0.0) - s * t + jnp.log1p(jnp.exp(-jnp.abs(s)))
        pt = p * t + (1.0 - p) * (1.0 - t)
        w = 0.25 * t + 0.75 * (1.0 - t)
        om = 1.0 - pt
        fsum = fsum + jnp.sum(w * om * om * ce).reshape(1, 1)

    out_ref[:, :] = (fsum * (1.0 / float(_B * _N))
                     + num_s[:, :] / jnp.maximum(den_s[:, :], 1.0))


def _run(predicted_locs, predicted_scores, boxes, labels, priors,
         interpret=False):
    B, N, C = predicted_scores.shape
    locs_t = jnp.pad(jnp.transpose(predicted_locs, (0, 2, 1)),
                     ((0, 0), (0, 0), (0, _PAD - N)))
    pri_t = jnp.pad(jnp.transpose(priors, (1, 0)), ((0, 0), (0, _PAD - N)))
    lvals = labels.astype(jnp.float32)[..., None]
    scores_t = jnp.transpose(predicted_scores, (0, 2, 1))

    out = pl.pallas_call(
        _fused_kernel,
        grid=(1,),
        in_specs=[
            pl.BlockSpec((B, 4, _PAD), lambda i: (0, 0, 0)),
            pl.BlockSpec((4, _PAD), lambda i: (0, 0)),
            pl.BlockSpec((B, _NOBJ, 4), lambda i: (0, 0, 0)),
            pl.BlockSpec((B, _NOBJ, 1), lambda i: (0, 0, 0)),
            pl.BlockSpec((B, _C, _N), lambda i: (0, 0, 0)),
        ],
        out_specs=pl.BlockSpec((1, 1), lambda i: (0, 0)),
        out_shape=jax.ShapeDtypeStruct((1, 1), jnp.float32),
        scratch_shapes=[
            pltpu.VMEM((_B, _N), jnp.float32),
            pltpu.VMEM((1, 1), jnp.float32),
            pltpu.VMEM((1, 1), jnp.float32),
        ],
        interpret=interpret,
    )(locs_t, pri_t, boxes, lvals, scores_t)
    return out[0, 0]


def kernel(predicted_locs, predicted_scores, boxes, labels, priors):
    return _run(predicted_locs, predicted_scores, boxes, labels, priors)


# speedup vs baseline: 1.5382x; 1.1251x over previous
import jax
import jax.numpy as jnp
from jax import lax
from jax.experimental import pallas as pl
from jax.experimental.pallas import tpu as pltpu

_N = 5460
_PAD = 5504
_C = 21
_B = 8
_NOBJ = 8
_NSEL = 49
_REGIONS = (
    (0, 4096, ((0, 4096, 9),)),
    (4096, 1024, ((0, 1024, 9),)),
    (5120, 256, ((0, 256, 9),)),
    (5376, 84, ((0, 64, 9), (64, 80, 9), (80, 84, 4))),
)


def _assignment(locs_ref, pri_ref, boxes_ref, lvals_ref,
                lab_s, num_s, den_s):
    px = pri_ref[0:1, :][None]
    py = pri_ref[1:2, :][None]
    pw = pri_ref[2:3, :][None]
    ph = pri_ref[3:4, :][None]

    bx0 = boxes_ref[:, :, 0:1]
    by0 = boxes_ref[:, :, 1:2]
    bx1 = boxes_ref[:, :, 2:3]
    by1 = boxes_ref[:, :, 3:4]
    cbx = (bx0 + bx1) / 2.0
    cby = (by0 + by1) / 2.0

    dist = jnp.sqrt((cbx - px) ** 2 + (cby - py) ** 2)

    qx0 = px - pw / 2.0
    qy0 = py - ph / 2.0
    qx1 = px + pw / 2.0
    qy1 = py + ph / 2.0

    ltx = jnp.maximum(bx0, qx0)
    lty = jnp.maximum(by0, qy0)
    rbx = jnp.minimum(bx1, qx1)
    rby = jnp.minimum(by1, qy1)
    inter = jnp.clip(rbx - ltx, 0.0, None) * jnp.clip(rby - lty, 0.0, None)
    area_a = (bx1 - bx0) * (by1 - by0)
    area_b = (qx1 - qx0) * (qy1 - qy0)
    ov = inter / jnp.clip(area_a + area_b - inter, 1e-9, None)

    chains = []
    for (start, width, subs) in _REGIONS:
        d_sl = dist[:, :, start:start + width]
        col = lax.broadcasted_iota(jnp.int32, (1, 1, width), 2)
        for (lo, hi, k) in subs:
            lvmask = (col >= lo) & (col < hi)
            d = jnp.where(lvmask, d_sl, jnp.inf)
            chains.append({"start": start, "k": k, "col": col,
                           "lvmask": lvmask, "d": d})
    for t in range(9):
        for ch in chains:
            if t >= ch["k"]:
                continue
            d = ch["d"]
            col = ch["col"]
            rm = jnp.min(d, axis=2, keepdims=True)
            cand = jnp.where(d == rm, col, jnp.int32(2 ** 30))
            idx = jnp.min(cand, axis=2, keepdims=True)
            ch["d"] = jnp.where(col == idx, jnp.inf, d)
    region_sel = {}
    for ch in chains:
        s = jnp.isinf(ch["d"]) & ch["lvmask"]
        key = ch["start"]
        region_sel[key] = (jnp.logical_or(region_sel[key], s)
                          if key in region_sel else s)
    sel = jnp.concatenate([region_sel[s] for (s, _, _) in _REGIONS], axis=2)

    self32 = sel.astype(jnp.float32)
    mean = jnp.sum(ov * self32, axis=2, keepdims=True) / float(_NSEL)
    dev = (ov - mean) * self32
    var = jnp.sum(dev * dev, axis=2, keepdims=True) / float(_NSEL - 1)
    thr = mean + jnp.sqrt(var)

    inside = (bx0 <= px) & (px <= bx1) & (by0 <= py) & (py <= by1)
    m = sel & (ov > thr) & inside

    li = lvals_ref[:, :, :].astype(jnp.int32)
    obcode = lax.broadcasted_iota(jnp.int32, (1, _NOBJ, 1), 1) * 32 + li
    q = jnp.where(m, obcode, 0)
    labcode = jnp.max(q, axis=1)
    lab_s[:, :] = (labcode & 31).astype(jnp.float32)

    g0 = locs_ref[:, 0:1, :]
    g1 = locs_ref[:, 1:2, :]
    g2 = locs_ref[:, 2:3, :]
    g3 = locs_ref[:, 3:4, :]
    dcx = g0 * pw / 10.0 + px
    dcy = g1 * ph / 10.0 + py
    dw = jnp.exp(g2 / 5.0) * pw
    dh = jnp.exp(g3 / 5.0) * ph
    dx0 = dcx - dw / 2.0
    dy0 = dcy - dh / 2.0
    dx1 = dcx + dw / 2.0
    dy1 = dcy + dh / 2.0

    ltx2 = jnp.maximum(dx0, bx0)
    lty2 = jnp.maximum(dy0, by0)
    rbx2 = jnp.minimum(dx1, bx1)
    rby2 = jnp.minimum(dy1, by1)
    inter2 = (jnp.clip(rbx2 - ltx2, 0.0, None)
              * jnp.clip(rby2 - lty2, 0.0, None))
    ap = (dx1 - dx0) * (dy1 - dy0)
    iou2 = inter2 / jnp.clip(ap + area_a - inter2, 1e-9, None)
    cpx = (dx0 + dx1) / 2.0
    cpy = (dy0 + dy1) / 2.0
    d2 = (cpx - cbx) ** 2 + (cpy - cby) ** 2
    ex0 = jnp.minimum(dx0, bx0)
    ey0 = jnp.minimum(dy0, by0)
    ex1 = jnp.maximum(dx1, bx1)
    ey1 = jnp.maximum(dy1, by1)
    c2 = (ex1 - ex0) ** 2 + (ey1 - ey0) ** 2
    dloss = 1.0 - iou2 + d2 / jnp.clip(c2, 1e-9, None)

    mf = m.astype(jnp.float32)
    num_s[:, :] = jnp.sum(dloss * mf).reshape(1, 1)
    den_s[:, :] = jnp.sum(mf).reshape(1, 1)


def _fused_kernel(locs_ref, pri_ref, boxes_ref, lvals_ref, sc_ref,
                  out_ref, lab_s, num_s, den_s):
    _assignment(locs_ref, pri_ref, boxes_ref, lvals_ref,
                lab_s, num_s, den_s)

    cls = lax.broadcasted_iota(jnp.int32, (_C, 1), 0).astype(jnp.float32)
    fsum = jnp.zeros((1, 1), jnp.float32)
    for i in range(_B):
        labv = lab_s[i:i + 1, :]
        s = sc_ref[i]
        t = (cls == labv).astype(jnp.float32)
        p = jax.nn.sigmoid(s)
        ce = jnp.maximum(s, 0.0) - s * t + jnp.log1p(jnp.exp(-jnp.abs(s)))
        pt = p * t + (1.0 - p) * (1.0 - t)
        w = 0.25 * t + 0.75 * (1.0 - t)
        om = 1.0 - pt
        fsum = fsum + jnp.sum(w * om * om * ce).reshape(1, 1)

    out_ref[:, :] = (fsum * (1.0 / float(_B * _N))
                     + num_s[:, :] / jnp.maximum(den_s[:, :], 1.0))


def _run(predicted_locs, predicted_scores, boxes, labels, priors,
         interpret=False):
    B, N, C = predicted_scores.shape
    locs_t = jnp.transpose(predicted_locs, (0, 2, 1))
    pri_t = jnp.transpose(priors, (1, 0))
    lvals = labels.astype(jnp.float32)[..., None]
    scores_t = jnp.transpose(predicted_scores, (0, 2, 1))

    out = pl.pallas_call(
        _fused_kernel,
        grid=(1,),
        in_specs=[
            pl.BlockSpec((B, 4, _N), lambda i: (0, 0, 0)),
            pl.BlockSpec((4, _N), lambda i: (0, 0)),
            pl.BlockSpec((B, _NOBJ, 4), lambda i: (0, 0, 0)),
            pl.BlockSpec((B, _NOBJ, 1), lambda i: (0, 0, 0)),
            pl.BlockSpec((B, _C, _N), lambda i: (0, 0, 0)),
        ],
        out_specs=pl.BlockSpec((1, 1), lambda i: (0, 0)),
        out_shape=jax.ShapeDtypeStruct((1, 1), jnp.float32),
        scratch_shapes=[
            pltpu.VMEM((_B, _N), jnp.float32),
            pltpu.VMEM((1, 1), jnp.float32),
            pltpu.VMEM((1, 1), jnp.float32),
        ],
        interpret=interpret,
    )(locs_t, pri_t, boxes, lvals, scores_t)
    return out[0, 0]


def kernel(predicted_locs, predicted_scores, boxes, labels, priors):
    return _run(predicted_locs, predicted_scores, boxes, labels, priors)


# speedup vs baseline: 1.5779x; 1.0258x over previous
import jax
import jax.numpy as jnp
from jax import lax
from jax.experimental import pallas as pl
from jax.experimental.pallas import tpu as pltpu

_N = 5460
_PAD = 5504
_C = 21
_B = 8
_NOBJ = 8
_NSEL = 49
_REGIONS = (
    (0, 4096, ((0, 4096, 9),)),
    (4096, 1024, ((0, 1024, 9),)),
    (5120, 256, ((0, 256, 9),)),
    (5376, 84, ((0, 64, 9), (64, 80, 9), (80, 84, 4))),
)


def _assignment(locs_ref, pri_ref, boxes_ref, lvals_ref,
                lab_s, num_s, den_s):
    px = pri_ref[0:1, :][None]
    py = pri_ref[1:2, :][None]
    pw = pri_ref[2:3, :][None]
    ph = pri_ref[3:4, :][None]

    bx0 = boxes_ref[:, :, 0:1]
    by0 = boxes_ref[:, :, 1:2]
    bx1 = boxes_ref[:, :, 2:3]
    by1 = boxes_ref[:, :, 3:4]
    cbx = (bx0 + bx1) / 2.0
    cby = (by0 + by1) / 2.0

    dist = jnp.sqrt((cbx - px) ** 2 + (cby - py) ** 2)

    qx0 = px - pw / 2.0
    qy0 = py - ph / 2.0
    qx1 = px + pw / 2.0
    qy1 = py + ph / 2.0

    ltx = jnp.maximum(bx0, qx0)
    lty = jnp.maximum(by0, qy0)
    rbx = jnp.minimum(bx1, qx1)
    rby = jnp.minimum(by1, qy1)
    inter = jnp.clip(rbx - ltx, 0.0, None) * jnp.clip(rby - lty, 0.0, None)
    area_a = (bx1 - bx0) * (by1 - by0)
    area_b = (qx1 - qx0) * (qy1 - qy0)
    ov = inter / jnp.clip(area_a + area_b - inter, 1e-9, None)

    chains = []
    for (start, width, subs) in _REGIONS:
        d_sl = dist[:, :, start:start + width]
        col = lax.broadcasted_iota(jnp.int32, (1, 1, width), 2)
        for (lo, hi, k) in subs:
            lvmask = (col >= lo) & (col < hi)
            d = jnp.where(lvmask, d_sl, jnp.inf)
            chains.append({"start": start, "k": k, "col": col,
                           "lvmask": lvmask, "d": d})
    for t in range(9):
        for ch in chains:
            if t >= ch["k"]:
                continue
            d = ch["d"]
            col = ch["col"]
            rm = jnp.min(d, axis=2, keepdims=True)
            cand = jnp.where(d == rm, col, jnp.int32(2 ** 30))
            idx = jnp.min(cand, axis=2, keepdims=True)
            ch["d"] = jnp.where(col == idx, jnp.inf, d)
    region_sel = {}
    for ch in chains:
        s = jnp.isinf(ch["d"]) & ch["lvmask"]
        key = ch["start"]
        region_sel[key] = (jnp.logical_or(region_sel[key], s)
                          if key in region_sel else s)
    sel = jnp.concatenate([region_sel[s] for (s, _, _) in _REGIONS], axis=2)

    self32 = sel.astype(jnp.float32)
    mean = jnp.sum(ov * self32, axis=2, keepdims=True) / float(_NSEL)
    dev = (ov - mean) * self32
    var = jnp.sum(dev * dev, axis=2, keepdims=True) / float(_NSEL - 1)
    thr = mean + jnp.sqrt(var)

    inside = (bx0 <= px) & (px <= bx1) & (by0 <= py) & (py <= by1)
    m = sel & (ov > thr) & inside

    li = lvals_ref[:, :, :].astype(jnp.int32)
    obcode = lax.broadcasted_iota(jnp.int32, (1, _NOBJ, 1), 1) * 32 + li
    q = jnp.where(m, obcode, 0)
    labcode = jnp.max(q, axis=1)
    lab_s[:, :] = (labcode & 31).astype(jnp.float32)

    g0 = locs_ref[:, 0:1, :]
    g1 = locs_ref[:, 1:2, :]
    g2 = locs_ref[:, 2:3, :]
    g3 = locs_ref[:, 3:4, :]
    dcx = g0 * pw / 10.0 + px
    dcy = g1 * ph / 10.0 + py
    dw = jnp.exp(g2 / 5.0) * pw
    dh = jnp.exp(g3 / 5.0) * ph
    dx0 = dcx - dw / 2.0
    dy0 = dcy - dh / 2.0
    dx1 = dcx + dw / 2.0
    dy1 = dcy + dh / 2.0

    ltx2 = jnp.maximum(dx0, bx0)
    lty2 = jnp.maximum(dy0, by0)
    rbx2 = jnp.minimum(dx1, bx1)
    rby2 = jnp.minimum(dy1, by1)
    inter2 = (jnp.clip(rbx2 - ltx2, 0.0, None)
              * jnp.clip(rby2 - lty2, 0.0, None))
    ap = (dx1 - dx0) * (dy1 - dy0)
    iou2 = inter2 / jnp.clip(ap + area_a - inter2, 1e-9, None)
    cpx = (dx0 + dx1) / 2.0
    cpy = (dy0 + dy1) / 2.0
    d2 = (cpx - cbx) ** 2 + (cpy - cby) ** 2
    ex0 = jnp.minimum(dx0, bx0)
    ey0 = jnp.minimum(dy0, by0)
    ex1 = jnp.maximum(dx1, bx1)
    ey1 = jnp.maximum(dy1, by1)
    c2 = (ex1 - ex0) ** 2 + (ey1 - ey0) ** 2
    dloss = 1.0 - iou2 + d2 / jnp.clip(c2, 1e-9, None)

    mf = m.astype(jnp.float32)
    num_s[:, :] = jnp.sum(dloss * mf).reshape(1, 1)
    den_s[:, :] = jnp.sum(mf).reshape(1, 1)


def _fused_kernel(locs_ref, pri_ref, boxes_ref, lvals_ref, sc_ref,
                  out_ref, lab_s, num_s, den_s):
    i = pl.program_id(0)

    @pl.when(i == 0)
    def _():
        _assignment(locs_ref, pri_ref, boxes_ref, lvals_ref,
                    lab_s, num_s, den_s)
        out_ref[:, :] = jnp.zeros_like(out_ref)

    cls = lax.broadcasted_iota(jnp.int32, (_C, 1), 0).astype(jnp.float32)
    labv = lab_s[pl.ds(i, 1), :]
    s = sc_ref[0]
    e = jnp.exp(-jnp.abs(s))
    r = 1.0 / (1.0 + e)
    ce0 = jnp.maximum(s, 0.0) + jnp.log1p(e)
    pneg = 1.0 - r
    pos = s >= 0.0
    p = jnp.where(pos, r, pneg)
    q = jnp.where(pos, pneg, r)
    base = (0.75 * ce0) * (p * p)
    hot = (0.25 * (ce0 - s)) * (q * q)
    val = jnp.where(cls == labv, hot, base)
    out_ref[:, :] = out_ref[:, :] + (jnp.sum(val)
                                     * (1.0 / float(_B * _N))).reshape(1, 1)

    @pl.when(i == _B - 1)
    def _():
        out_ref[:, :] = (out_ref[:, :]
                         + num_s[:, :] / jnp.maximum(den_s[:, :], 1.0))


def _run(predicted_locs, predicted_scores, boxes, labels, priors,
         interpret=False):
    B, N, C = predicted_scores.shape
    locs_t = jnp.transpose(predicted_locs, (0, 2, 1))
    pri_t = jnp.transpose(priors, (1, 0))
    lvals = labels.astype(jnp.float32)[..., None]
    scores_t = jnp.transpose(predicted_scores, (0, 2, 1))

    out = pl.pallas_call(
        _fused_kernel,
        grid=(B,),
        in_specs=[
            pl.BlockSpec((B, 4, _N), lambda i: (0, 0, 0)),
            pl.BlockSpec((4, _N), lambda i: (0, 0)),
            pl.BlockSpec((B, _NOBJ, 4), lambda i: (0, 0, 0)),
            pl.BlockSpec((B, _NOBJ, 1), lambda i: (0, 0, 0)),
            pl.BlockSpec((1, _C, _N), lambda i: (i, 0, 0)),
        ],
        out_specs=pl.BlockSpec((1, 1), lambda i: (0, 0)),
        out_shape=jax.ShapeDtypeStruct((1, 1), jnp.float32),
        scratch_shapes=[
            pltpu.VMEM((_B, _N), jnp.float32),
            pltpu.VMEM((1, 1), jnp.float32),
            pltpu.VMEM((1, 1), jnp.float32),
        ],
        interpret=interpret,
    )(locs_t, pri_t, boxes, lvals, scores_t)
    return out[0, 0]


def kernel(predicted_locs, predicted_scores, boxes, labels, priors):
    return _run(predicted_locs, predicted_scores, boxes, labels, priors)
